# Initial kernel scaffold; baseline (speedup 1.0000x reference)
import functools

import jax
import jax.numpy as jnp
from jax import lax
from jax.experimental import pallas as pl
from jax.experimental.pallas import tpu as pltpu
from jax.experimental.pallas import tpu_sc as plsc

BATCH = 16384
HIST = 50
EMBED = 64
TOTAL = BATCH * HIST
NUM_WORKERS = 32
PER_WORKER = TOTAL // NUM_WORKERS
CHUNK = 128
N_CHUNKS = PER_WORKER // CHUNK
NBUF = 8


def _make_gather():
    mesh = plsc.VectorSubcoreMesh(core_axis_name="c", subcore_axis_name="s")

    @functools.partial(
        pl.kernel,
        mesh=mesh,
        out_type=jax.ShapeDtypeStruct((TOTAL, EMBED), jnp.float32),
        scratch_types=[
            pltpu.VMEM((N_CHUNKS, CHUNK), jnp.int32),
            pltpu.VMEM((NBUF, CHUNK, EMBED), jnp.float32),
            pltpu.SemaphoreType.DMA((NBUF,)),
            pltpu.SemaphoreType.DMA((NBUF,)),
        ],
    )
    def gather(table_hbm, idx_hbm, out_hbm, idx_v, rows_v, gsem, osem):
        wid = lax.axis_index("s") * 2 + lax.axis_index("c")
        out_base = wid * PER_WORKER

        pltpu.sync_copy(idx_hbm.at[wid], idx_v)

        def fire_gather(c, b):
            pltpu.async_copy(table_hbm.at[idx_v.at[c]], rows_v.at[b], gsem.at[b])

        for b in range(NBUF):
            fire_gather(b, b)

        def body(g, carry):
            c0 = g * NBUF
            store_descs = []
            for b in range(NBUF):
                c = c0 + b
                pltpu.make_async_copy(
                    table_hbm.at[pl.ds(0, CHUNK)], rows_v.at[b], gsem.at[b]
                ).wait()
                d = pltpu.make_async_copy(
                    rows_v.at[b],
                    out_hbm.at[pl.ds(out_base + c * CHUNK, CHUNK)],
                    osem.at[b],
                )
                d.start()
                store_descs.append(d)
            for b in range(NBUF):
                store_descs[b].wait()
                c_next = c0 + b + NBUF

                @pl.when(c_next < N_CHUNKS)
                def _():
                    fire_gather(c_next, b)

            return carry

        lax.fori_loop(0, N_CHUNKS // NBUF, body, 0)

    return gather


_gather_rows = _make_gather()


def kernel(gps_idx, weight):
    idx = gps_idx.reshape(NUM_WORKERS, N_CHUNKS, CHUNK).astype(jnp.int32)
    out = _gather_rows(weight, idx)
    return out.reshape(BATCH, HIST, EMBED)


# speedup vs baseline: 1.8716x; 1.8716x over previous
import functools

import jax
import jax.numpy as jnp
from jax import lax
from jax.experimental import pallas as pl
from jax.experimental.pallas import tpu as pltpu
from jax.experimental.pallas import tpu_sc as plsc

BATCH = 16384
HIST = 50
EMBED = 64
TOTAL = BATCH * HIST
NUM_WORKERS = 32
PER_WORKER = TOTAL // NUM_WORKERS
CHUNK = 128
N_CHUNKS = PER_WORKER // CHUNK
NBUF = 8


def _make_gather():
    mesh = plsc.VectorSubcoreMesh(core_axis_name="c", subcore_axis_name="s")

    @functools.partial(
        pl.kernel,
        mesh=mesh,
        compiler_params=pltpu.CompilerParams(use_tc_tiling_on_sc=False),
        out_type=jax.ShapeDtypeStruct((TOTAL, EMBED), jnp.float32),
        scratch_types=[
            pltpu.VMEM((N_CHUNKS, CHUNK), jnp.int32),
            pltpu.VMEM((NBUF, CHUNK, EMBED), jnp.float32),
            pltpu.SemaphoreType.DMA((NBUF,)),
            pltpu.SemaphoreType.DMA((NBUF,)),
        ],
    )
    def gather(table_hbm, idx_hbm, out_hbm, idx_v, rows_v, gsem, osem):
        wid = lax.axis_index("s") * 2 + lax.axis_index("c")
        out_base = wid * PER_WORKER

        pltpu.sync_copy(idx_hbm.at[wid], idx_v)

        def fire_gather(c, b):
            pltpu.async_copy(table_hbm.at[idx_v.at[c]], rows_v.at[b], gsem.at[b])

        for b in range(NBUF):
            fire_gather(b, b)

        def body(g, carry):
            c0 = g * NBUF
            store_descs = []
            for b in range(NBUF):
                c = c0 + b
                pltpu.make_async_copy(
                    table_hbm.at[pl.ds(0, CHUNK)], rows_v.at[b], gsem.at[b]
                ).wait()
                d = pltpu.make_async_copy(
                    rows_v.at[b],
                    out_hbm.at[pl.ds(out_base + c * CHUNK, CHUNK)],
                    osem.at[b],
                )
                d.start()
                store_descs.append(d)
            for b in range(NBUF):
                store_descs[b].wait()
                c_next = c0 + b + NBUF

                @pl.when(c_next < N_CHUNKS)
                def _():
                    fire_gather(c_next, b)

            return carry

        lax.fori_loop(0, N_CHUNKS // NBUF, body, 0)

    return gather


_gather_rows = _make_gather()


def kernel(gps_idx, weight):
    idx = gps_idx.reshape(NUM_WORKERS, N_CHUNKS, CHUNK).astype(jnp.int32)
    out = _gather_rows(weight, idx)
    return out.reshape(BATCH, HIST, EMBED)
